# baseline (speedup 1.0000x reference)
import functools

import jax
import jax.numpy as jnp
from jax import lax
from jax.experimental import pallas as pl
from jax.experimental.pallas import tpu as pltpu
from jax.experimental.pallas import tpu_sc as plsc

G = 64
NC = 2
NS = 16
NW = NC * NS
BLK = 2000
KC = 2000
K = 80


def _mesh():
    return plsc.VectorSubcoreMesh(
        core_axis_name="c", subcore_axis_name="s", num_cores=NC, num_subcores=NS)


def _emb_m_body(x_ref, w1_ref, b1_ref, w2_ref, b2_ref, rw_ref, oh_ref, om_ref):
    t = jnp.dot(x_ref[...], w1_ref[...], preferred_element_type=jnp.float32)
    t = jnp.maximum(t + b1_ref[...], 0.0)
    h = jnp.dot(t, w2_ref[...], preferred_element_type=jnp.float32) + b2_ref[...]
    oh_ref[...] = h
    for r in range(om_ref.shape[0]):
        om_ref[r] = jnp.dot(h, rw_ref[r], preferred_element_type=jnp.float32)


def _upd_m_body(h_ref, w_ref, b_ref, p0_ref, p1_ref, rw_ref, oh_ref, om_ref):
    v = jnp.dot(h_ref[...], w_ref[...], preferred_element_type=jnp.float32)
    v = jnp.maximum(v + b_ref[...] + p0_ref[0] + p1_ref[0], 0.0)
    oh_ref[...] = v
    for r in range(om_ref.shape[0]):
        om_ref[r] = jnp.dot(v, rw_ref[r], preferred_element_type=jnp.float32)


def _upd_pool_body(h_ref, w_ref, b_ref, p0_ref, p1_ref, bt_ref,
                   hw1_ref, hb1_ref, hw2_ref, hb2_ref, o_ref, acc_ref):
    i = pl.program_id(0)

    @pl.when(i == 0)
    def _():
        acc_ref[...] = jnp.zeros_like(acc_ref)

    v = jnp.dot(h_ref[...], w_ref[...], preferred_element_type=jnp.float32)
    v = v + b_ref[...] + p0_ref[0] + p1_ref[0]
    bvec = bt_ref[0]
    oh = (lax.broadcasted_iota(jnp.int32, (G, bvec.shape[1]), 0) == bvec)
    acc_ref[...] += jnp.dot(oh.astype(jnp.float32), v,
                            preferred_element_type=jnp.float32)

    @pl.when(i == pl.num_programs(0) - 1)
    def _():
        p = acc_ref[...]
        t = jnp.maximum(
            jnp.dot(p, hw1_ref[...], preferred_element_type=jnp.float32) + hb1_ref[...], 0.0)
        o_ref[...] = jnp.dot(t, hw2_ref[...], preferred_element_type=jnp.float32) + hb2_ref[...]


def _make_prep(E, R, NP, CN):
    EP = E // NW

    @functools.partial(
        pl.kernel,
        out_type=(jax.ShapeDtypeStruct((E,), jnp.int32),
                  jax.ShapeDtypeStruct((E,), jnp.int32),
                  jax.ShapeDtypeStruct((NW, CN), jnp.float32)),
        mesh=_mesh(),
        compiler_params=pltpu.CompilerParams(needs_layout_passes=False),
        scratch_types=[
            pltpu.VMEM((KC,), jnp.int32),
            pltpu.VMEM((KC,), jnp.int32),
            pltpu.VMEM((KC,), jnp.int32),
            pltpu.VMEM((KC,), jnp.int32),
            pltpu.VMEM((CN,), jnp.float32),
        ] + [pltpu.VMEM((KC,), jnp.float32) for _ in range(R)],
    )
    def prep(src_hbm, dst_hbm, attr_hbm, gidx_hbm, sidx_hbm, cnt_hbm,
             s_v, d_v, gi_v, si_v, cnt_v, *a_refs):
        cid = lax.axis_index("c")
        sid = lax.axis_index("s")
        w = cid * NS + sid
        ones = jnp.ones((16,), jnp.float32)

        def zero(i, _):
            cnt_v[pl.ds(i * 16, 16)] = jnp.zeros((16,), jnp.float32)
            return 0
        lax.fori_loop(0, CN // 16, zero, 0)

        def chunk(ci, _):
            base = w * EP + ci * KC
            pltpu.sync_copy(src_hbm.at[pl.ds(base, KC)], s_v)
            pltpu.sync_copy(dst_hbm.at[pl.ds(base, KC)], d_v)
            for r in range(R):
                pltpu.sync_copy(attr_hbm.at[pl.ds(r * E + base, KC)], a_refs[r])

            def grp(j, _):
                off = j * 16
                best = a_refs[0][pl.ds(off, 16)]
                t = jnp.zeros((16,), jnp.int32)
                for r in range(1, R):
                    ar = a_refs[r][pl.ds(off, 16)]
                    m = ar > best
                    t = jnp.where(m, r, t)
                    best = jnp.where(m, ar, best)
                sv = s_v[pl.ds(off, 16)]
                dv = d_v[pl.ds(off, 16)]
                gi_v[pl.ds(off, 16)] = t * NP + sv
                si = dv * R + t
                si_v[pl.ds(off, 16)] = si
                plsc.addupdate_scatter(cnt_v, [si], ones)
                return 0
            lax.fori_loop(0, KC // 16, grp, 0)

            pltpu.sync_copy(gi_v, gidx_hbm.at[pl.ds(base, KC)])
            pltpu.sync_copy(si_v, sidx_hbm.at[pl.ds(base, KC)])
            return 0
        lax.fori_loop(0, EP // KC, chunk, 0)

        pltpu.sync_copy(cnt_v, cnt_hbm.at[w])

    return prep


def _make_scale(E, NR, CN):
    EP = E // NW
    SW = CN // NW

    @functools.partial(
        pl.kernel,
        out_type=jax.ShapeDtypeStruct((E,), jnp.float32),
        mesh=_mesh(),
        compiler_params=pltpu.CompilerParams(needs_layout_passes=False),
        scratch_types=[
            pltpu.VMEM((KC,), jnp.int32),
            pltpu.VMEM((KC,), jnp.float32),
            pltpu.VMEM((NR,), jnp.float32),
            pltpu.VMEM((NW, SW), jnp.float32),
            pltpu.VMEM((SW,), jnp.float32),
            pltpu.VMEM_SHARED((CN,), jnp.float32),
        ],
    )
    def scale(sidx_hbm, cnt_hbm, sc_hbm, si_v, sc_v, inv_v, parts_v, ist_v, inv_sh):
        cid = lax.axis_index("c")
        sid = lax.axis_index("s")
        w = cid * NS + sid

        for half in range(NW // NS):
            soff = (half * NS + sid) * SW
            pltpu.sync_copy(cnt_hbm.at[pl.ds(0, NW), pl.ds(soff, SW)], parts_v)

            def red(g, _):
                off = g * 16
                s = jnp.zeros((16,), jnp.float32)
                for p in range(NW):
                    s = s + parts_v[p, pl.ds(off, 16)]
                ist_v[pl.ds(off, 16)] = 1.0 / jnp.maximum(s, 1.0)
                return 0
            lax.fori_loop(0, SW // 16, red, 0)
            pltpu.sync_copy(ist_v, inv_sh.at[pl.ds(soff, SW)])
        plsc.subcore_barrier()
        pltpu.sync_copy(inv_sh.at[pl.ds(0, NR)], inv_v)

        def chunk(ci, _):
            base = w * EP + ci * KC
            pltpu.sync_copy(sidx_hbm.at[pl.ds(base, KC)], si_v)

            def grp(j, _):
                off = j * 16
                si = si_v[pl.ds(off, 16)]
                sc_v[pl.ds(off, 16)] = plsc.load_gather(inv_v, [si])
                return 0
            lax.fori_loop(0, KC // 16, grp, 0)

            pltpu.sync_copy(sc_v, sc_hbm.at[pl.ds(base, KC)])
            return 0
        lax.fori_loop(0, EP // KC, chunk, 0)

    return scale


def _make_edge(E, NA, H):
    EP = E // NW
    STRIPE = NA // NS
    NCHUNK = EP // K
    NBUF = 3
    NTRI = NCHUNK // NBUF
    REM = NCHUNK - NTRI * NBUF
    assert EP % K == 0

    @functools.partial(
        pl.kernel,
        out_type=jax.ShapeDtypeStruct((NC, NA, H), jnp.float32),
        mesh=_mesh(),
        compiler_params=pltpu.CompilerParams(needs_layout_passes=False),
        scratch_types=(
            [pltpu.VMEM((K, H), jnp.float32)] * NBUF
            + [pltpu.VMEM((K,), jnp.int32)] * NBUF
            + [pltpu.VMEM((K,), jnp.float32)] * NBUF
            + [pltpu.VMEM((EP,), jnp.int32)]
            + [pltpu.VMEM_SHARED((NA, H), jnp.float32)]
            + [pltpu.SemaphoreType.DMA] * (2 * NBUF)
        ),
    )
    def edge(m_hbm, gidx_hbm, dst_hbm, sce_hbm, out_hbm, *scr):
        rows = scr[0:NBUF]
        dbuf = scr[NBUF:2 * NBUF]
        scb = scr[2 * NBUF:3 * NBUF]
        gi_all = scr[3 * NBUF]
        acc_sh = scr[3 * NBUF + 1]
        gsem = scr[3 * NBUF + 2:3 * NBUF + 2 + NBUF]
        wsem = scr[3 * NBUF + 2 + NBUF:]
        cid = lax.axis_index("c")
        sid = lax.axis_index("s")
        w = cid * NS + sid
        ebase = w * EP

        def zrow(i, _):
            for c in range(H // 16):
                rows[0][i, pl.ds(c * 16, 16)] = jnp.zeros((16,), jnp.float32)
            return 0
        lax.fori_loop(0, K, zrow, 0)
        for b in range(STRIPE // K):
            pltpu.sync_copy(rows[0], acc_sh.at[pl.ds(sid * STRIPE + b * K, K)])
        rem = STRIPE % K
        if rem:
            pltpu.sync_copy(rows[0].at[pl.ds(0, rem)],
                            acc_sh.at[pl.ds(sid * STRIPE + (STRIPE // K) * K, rem)])
        pltpu.sync_copy(gidx_hbm.at[pl.ds(ebase, EP)], gi_all)
        plsc.subcore_barrier()

        def g_desc(c, p):
            return pltpu.make_async_copy(
                m_hbm.at[gi_all.at[pl.ds(c * K, K)]], rows[p], gsem[p])

        def d_desc(c, p):
            return pltpu.make_async_copy(
                dst_hbm.at[pl.ds(ebase + c * K, K)], dbuf[p], gsem[p])

        def s_desc(c, p):
            return pltpu.make_async_copy(
                sce_hbm.at[pl.ds(ebase + c * K, K)], scb[p], gsem[p])

        def w_desc(p):
            return pltpu.make_async_copy(rows[p], acc_sh.at[dbuf[p]], wsem[p])

        def start(c, p):
            g_desc(c, p).start()
            d_desc(c, p).start()
            s_desc(c, p).start()

        def wait_g(c, p):
            g_desc(c, p).wait()
            d_desc(c, p).wait()
            s_desc(c, p).wait()

        def process(p):
            rb = rows[p]
            sb = scb[p]

            def mj(j2, _):
                off = j2 * 16
                sv = sb[pl.ds(off, 16)]
                for jj in range(16):
                    s = sv[jj]
                    row = off + jj
                    for cc in range(H // 16):
                        rb[row, pl.ds(cc * 16, 16)] = rb[row, pl.ds(cc * 16, 16)] * s
                return 0
            lax.fori_loop(0, K // 16, mj, 0)

        def step(cc, q, may_issue=True):
            qr = (q - 2) % NBUF

            @pl.when(cc >= 2)
            def _():
                w_desc(qr).wait()
            if may_issue:
                @pl.when(jnp.logical_and(cc >= 2, cc + 1 < NCHUNK))
                def _():
                    start(cc + 1, qr)
            wait_g(cc, q)
            process(q)
            w_desc(q).start(add=True)

        for q in range(NBUF):
            start(q, q)

        def tri(i3, _):
            c = NBUF * i3
            for q in range(NBUF):
                step(c + q, q)
            return 0
        lax.fori_loop(0, NTRI, tri, 0)

        for q in range(REM):
            cc = NTRI * NBUF + q
            step(cc, q, may_issue=(cc + 1 < NCHUNK))
        for cc in (NCHUNK - 2, NCHUNK - 1):
            w_desc(cc % NBUF).wait()

        plsc.subcore_barrier()
        pltpu.sync_copy(acc_sh.at[pl.ds(sid * STRIPE, STRIPE)],
                        out_hbm.at[cid, pl.ds(sid * STRIPE, STRIPE)])

    return edge


def kernel(x, edge_index, edge_attr, batch, emb_W1, emb_b1, emb_W2, emb_b2,
           rel_w, root_w, conv_b, head_W1, head_b1, head_W2, head_b2):
    N, D = x.shape
    E = edge_index.shape[1]
    R = edge_attr.shape[1]
    H = emb_W1.shape[1]
    OUT = head_W2.shape[1]
    DEPTH = rel_w.shape[0]
    NP = N
    CN = -(-R * N // (NW * 128)) * (NW * 128)

    full = lambda shape: pl.BlockSpec(shape, lambda *_: tuple(0 for _ in shape))
    rowb = pl.BlockSpec((BLK, H), lambda i: (i, 0))
    mblk = pl.BlockSpec((R, BLK, H), lambda i: (0, i, 0))
    pblk0 = pl.BlockSpec((1, BLK, H), lambda i: (0, i, 0))
    pblk1 = pl.BlockSpec((1, BLK, H), lambda i: (1, i, 0))

    src = edge_index[0]
    dst = edge_index[1]
    attr_cm = edge_attr.T.reshape(-1)
    gidx, sidx, cnt_parts = _make_prep(E, R, NP, CN)(src, dst, attr_cm)
    sc_e = _make_scale(E, R * N, CN)(sidx, cnt_parts)

    h, m = pl.pallas_call(
        _emb_m_body,
        grid=(NP // BLK,),
        in_specs=[pl.BlockSpec((BLK, D), lambda i: (i, 0)), full((D, H)),
                  full((1, H)), full((H, H)), full((1, H)), full((R, H, H))],
        out_specs=[rowb, mblk],
        out_shape=[jax.ShapeDtypeStruct((NP, H), jnp.float32),
                   jax.ShapeDtypeStruct((R, NP, H), jnp.float32)],
    )(x, emb_W1, emb_b1.reshape(1, H), emb_W2, emb_b2.reshape(1, H), rel_w[0])

    NA = -(-N // 128) * 128
    edge_call = _make_edge(E, NA, H)

    for l in range(DEPTH):
        parts = edge_call(m.reshape(R * NP, H), gidx, dst, sc_e)

        if l != DEPTH - 1:
            h, m = pl.pallas_call(
                _upd_m_body,
                grid=(NP // BLK,),
                in_specs=[rowb, full((H, H)), full((1, H)), pblk0, pblk1,
                          full((R, H, H))],
                out_specs=[rowb, mblk],
                out_shape=[jax.ShapeDtypeStruct((NP, H), jnp.float32),
                           jax.ShapeDtypeStruct((R, NP, H), jnp.float32)],
            )(h, root_w[l], conv_b[l].reshape(1, H), parts, parts,
              rel_w[l + 1])
        else:
            out = pl.pallas_call(
                _upd_pool_body,
                grid=(NP // BLK,),
                in_specs=[rowb, full((H, H)), full((1, H)), pblk0, pblk1,
                          pl.BlockSpec((1, 1, BLK), lambda i: (i, 0, 0)),
                          full((H, H)), full((1, H)), full((H, OUT)),
                          full((1, OUT))],
                out_specs=full((G, OUT)),
                out_shape=jax.ShapeDtypeStruct((G, OUT), jnp.float32),
                scratch_shapes=[pltpu.VMEM((G, H), jnp.float32)],
            )(h, root_w[l], conv_b[l].reshape(1, H), parts, parts,
              batch.reshape(NP // BLK, 1, BLK),
              head_W1, head_b1.reshape(1, H), head_W2, head_b2.reshape(1, OUT))

    return out


# speedup vs baseline: 1.0541x; 1.0541x over previous
import functools

import jax
import jax.numpy as jnp
from jax import lax
from jax.experimental import pallas as pl
from jax.experimental.pallas import tpu as pltpu
from jax.experimental.pallas import tpu_sc as plsc

G = 64
NC = 2
NS = 16
NW = NC * NS
BLK = 2000
KC = 2000
K = 80


def _mesh():
    return plsc.VectorSubcoreMesh(
        core_axis_name="c", subcore_axis_name="s", num_cores=NC, num_subcores=NS)


def _emb_m_body(x_ref, w1_ref, b1_ref, w2_ref, b2_ref, rw_ref, oh_ref, om_ref):
    t = jnp.dot(x_ref[...], w1_ref[...], preferred_element_type=jnp.float32)
    t = jnp.maximum(t + b1_ref[...], 0.0)
    h = jnp.dot(t, w2_ref[...], preferred_element_type=jnp.float32) + b2_ref[...]
    oh_ref[...] = h
    for r in range(om_ref.shape[0]):
        om_ref[r] = jnp.dot(h, rw_ref[r], preferred_element_type=jnp.float32)


def _upd_m_body(h_ref, w_ref, b_ref, p0_ref, p1_ref, rw_ref, oh_ref, om_ref):
    v = jnp.dot(h_ref[...], w_ref[...], preferred_element_type=jnp.float32)
    v = jnp.maximum(v + b_ref[...] + p0_ref[0] + p1_ref[0], 0.0)
    oh_ref[...] = v
    for r in range(om_ref.shape[0]):
        om_ref[r] = jnp.dot(v, rw_ref[r], preferred_element_type=jnp.float32)


def _upd_pool_body(h_ref, w_ref, b_ref, p0_ref, p1_ref, bt_ref,
                   hw1_ref, hb1_ref, hw2_ref, hb2_ref, o_ref, acc_ref):
    i = pl.program_id(0)

    @pl.when(i == 0)
    def _():
        acc_ref[...] = jnp.zeros_like(acc_ref)

    v = jnp.dot(h_ref[...], w_ref[...], preferred_element_type=jnp.float32)
    v = v + b_ref[...] + p0_ref[0] + p1_ref[0]
    bvec = bt_ref[0]
    oh = (lax.broadcasted_iota(jnp.int32, (G, bvec.shape[1]), 0) == bvec)
    acc_ref[...] += jnp.dot(oh.astype(jnp.float32), v,
                            preferred_element_type=jnp.float32)

    @pl.when(i == pl.num_programs(0) - 1)
    def _():
        p = acc_ref[...]
        t = jnp.maximum(
            jnp.dot(p, hw1_ref[...], preferred_element_type=jnp.float32) + hb1_ref[...], 0.0)
        o_ref[...] = jnp.dot(t, hw2_ref[...], preferred_element_type=jnp.float32) + hb2_ref[...]


def _make_prep(E, R, NP, CN):
    EP = E // NW

    @functools.partial(
        pl.kernel,
        out_type=(jax.ShapeDtypeStruct((E,), jnp.int32),
                  jax.ShapeDtypeStruct((E,), jnp.int32),
                  jax.ShapeDtypeStruct((NW, CN), jnp.float32)),
        mesh=_mesh(),
        compiler_params=pltpu.CompilerParams(needs_layout_passes=False),
        scratch_types=[
            pltpu.VMEM((EP,), jnp.int32),
            pltpu.VMEM((EP,), jnp.int32),
            pltpu.VMEM((EP,), jnp.int32),
            pltpu.VMEM((EP,), jnp.int32),
            pltpu.VMEM((CN,), jnp.float32),
            pltpu.SemaphoreType.DMA,
        ] + [pltpu.VMEM((EP,), jnp.float32) for _ in range(R)],
    )
    def prep(src_hbm, dst_hbm, attr_hbm, gidx_hbm, sidx_hbm, cnt_hbm,
             s_v, d_v, gi_v, si_v, cnt_v, sem, *a_refs):
        cid = lax.axis_index("c")
        sid = lax.axis_index("s")
        w = cid * NS + sid
        ones = jnp.ones((16,), jnp.float32)
        base = w * EP

        descs = ([pltpu.make_async_copy(src_hbm.at[pl.ds(base, EP)], s_v, sem),
                  pltpu.make_async_copy(dst_hbm.at[pl.ds(base, EP)], d_v, sem)]
                 + [pltpu.make_async_copy(
                        attr_hbm.at[pl.ds(r * E + base, EP)], a_refs[r], sem)
                    for r in range(R)])
        for dsc in descs:
            dsc.start()

        def zero(i, _):
            cnt_v[pl.ds(i * 16, 16)] = jnp.zeros((16,), jnp.float32)
            return 0
        lax.fori_loop(0, CN // 16, zero, 0)
        for dsc in descs:
            dsc.wait()

        def grp(j, _):
            off = j * 16
            best = a_refs[0][pl.ds(off, 16)]
            t = jnp.zeros((16,), jnp.int32)
            for r in range(1, R):
                ar = a_refs[r][pl.ds(off, 16)]
                m = ar > best
                t = jnp.where(m, r, t)
                best = jnp.where(m, ar, best)
            sv = s_v[pl.ds(off, 16)]
            dv = d_v[pl.ds(off, 16)]
            gi_v[pl.ds(off, 16)] = t * NP + sv
            si = dv * R + t
            si_v[pl.ds(off, 16)] = si
            plsc.addupdate_scatter(cnt_v, [si], ones)
            return 0
        lax.fori_loop(0, EP // 16, grp, 0)

        pltpu.sync_copy(gi_v, gidx_hbm.at[pl.ds(base, EP)])
        pltpu.sync_copy(si_v, sidx_hbm.at[pl.ds(base, EP)])
        pltpu.sync_copy(cnt_v, cnt_hbm.at[w])

    return prep


def _make_scale(E, NR, CN):
    EP = E // NW
    SW = CN // NW

    @functools.partial(
        pl.kernel,
        out_type=jax.ShapeDtypeStruct((E,), jnp.float32),
        mesh=_mesh(),
        compiler_params=pltpu.CompilerParams(needs_layout_passes=False),
        scratch_types=[
            pltpu.VMEM((KC,), jnp.int32),
            pltpu.VMEM((KC,), jnp.float32),
            pltpu.VMEM((NR,), jnp.float32),
            pltpu.VMEM((NW, SW), jnp.float32),
            pltpu.VMEM((SW,), jnp.float32),
            pltpu.VMEM_SHARED((CN,), jnp.float32),
        ],
    )
    def scale(sidx_hbm, cnt_hbm, sc_hbm, si_v, sc_v, inv_v, parts_v, ist_v, inv_sh):
        cid = lax.axis_index("c")
        sid = lax.axis_index("s")
        w = cid * NS + sid

        for half in range(NW // NS):
            soff = (half * NS + sid) * SW
            pltpu.sync_copy(cnt_hbm.at[pl.ds(0, NW), pl.ds(soff, SW)], parts_v)

            def red(g, _):
                off = g * 16
                s = jnp.zeros((16,), jnp.float32)
                for p in range(NW):
                    s = s + parts_v[p, pl.ds(off, 16)]
                ist_v[pl.ds(off, 16)] = 1.0 / jnp.maximum(s, 1.0)
                return 0
            lax.fori_loop(0, SW // 16, red, 0)
            pltpu.sync_copy(ist_v, inv_sh.at[pl.ds(soff, SW)])
        plsc.subcore_barrier()
        pltpu.sync_copy(inv_sh.at[pl.ds(0, NR)], inv_v)

        def chunk(ci, _):
            base = w * EP + ci * KC
            pltpu.sync_copy(sidx_hbm.at[pl.ds(base, KC)], si_v)

            def grp(j, _):
                off = j * 16
                si = si_v[pl.ds(off, 16)]
                sc_v[pl.ds(off, 16)] = plsc.load_gather(inv_v, [si])
                return 0
            lax.fori_loop(0, KC // 16, grp, 0)

            pltpu.sync_copy(sc_v, sc_hbm.at[pl.ds(base, KC)])
            return 0
        lax.fori_loop(0, EP // KC, chunk, 0)

    return scale


def _make_edge(E, NA, H):
    EP = E // NW
    STRIPE = NA // NS
    NCHUNK = EP // K
    NBUF = 3
    NTRI = NCHUNK // NBUF
    REM = NCHUNK - NTRI * NBUF
    assert EP % K == 0

    @functools.partial(
        pl.kernel,
        out_type=jax.ShapeDtypeStruct((NC, NA, H), jnp.float32),
        mesh=_mesh(),
        compiler_params=pltpu.CompilerParams(needs_layout_passes=False),
        scratch_types=(
            [pltpu.VMEM((K, H), jnp.float32)] * NBUF
            + [pltpu.VMEM((K,), jnp.int32)] * NBUF
            + [pltpu.VMEM((K,), jnp.float32)] * NBUF
            + [pltpu.VMEM((EP,), jnp.int32)]
            + [pltpu.VMEM_SHARED((NA, H), jnp.float32)]
            + [pltpu.SemaphoreType.DMA] * (2 * NBUF)
        ),
    )
    def edge(m_hbm, gidx_hbm, dst_hbm, sce_hbm, out_hbm, *scr):
        rows = scr[0:NBUF]
        dbuf = scr[NBUF:2 * NBUF]
        scb = scr[2 * NBUF:3 * NBUF]
        gi_all = scr[3 * NBUF]
        acc_sh = scr[3 * NBUF + 1]
        gsem = scr[3 * NBUF + 2:3 * NBUF + 2 + NBUF]
        wsem = scr[3 * NBUF + 2 + NBUF:]
        cid = lax.axis_index("c")
        sid = lax.axis_index("s")
        w = cid * NS + sid
        ebase = w * EP

        def zrow(i, _):
            for c in range(H // 16):
                rows[0][i, pl.ds(c * 16, 16)] = jnp.zeros((16,), jnp.float32)
            return 0
        lax.fori_loop(0, K, zrow, 0)
        for b in range(STRIPE // K):
            pltpu.sync_copy(rows[0], acc_sh.at[pl.ds(sid * STRIPE + b * K, K)])
        rem = STRIPE % K
        if rem:
            pltpu.sync_copy(rows[0].at[pl.ds(0, rem)],
                            acc_sh.at[pl.ds(sid * STRIPE + (STRIPE // K) * K, rem)])
        pltpu.sync_copy(gidx_hbm.at[pl.ds(ebase, EP)], gi_all)
        plsc.subcore_barrier()

        def g_desc(c, p):
            return pltpu.make_async_copy(
                m_hbm.at[gi_all.at[pl.ds(c * K, K)]], rows[p], gsem[p])

        def d_desc(c, p):
            return pltpu.make_async_copy(
                dst_hbm.at[pl.ds(ebase + c * K, K)], dbuf[p], gsem[p])

        def s_desc(c, p):
            return pltpu.make_async_copy(
                sce_hbm.at[pl.ds(ebase + c * K, K)], scb[p], gsem[p])

        def w_desc(p):
            return pltpu.make_async_copy(rows[p], acc_sh.at[dbuf[p]], wsem[p])

        def start(c, p):
            g_desc(c, p).start()
            d_desc(c, p).start()
            s_desc(c, p).start()

        def wait_g(c, p):
            g_desc(c, p).wait()
            d_desc(c, p).wait()
            s_desc(c, p).wait()

        def process(p):
            rb = rows[p]
            sb = scb[p]

            def mj(j2, _):
                off = j2 * 16
                sv = sb[pl.ds(off, 16)]
                for jj in range(16):
                    s = sv[jj]
                    row = off + jj
                    for cc in range(H // 16):
                        rb[row, pl.ds(cc * 16, 16)] = rb[row, pl.ds(cc * 16, 16)] * s
                return 0
            lax.fori_loop(0, K // 16, mj, 0)

        def step(cc, q, may_issue=True):
            qr = (q - 2) % NBUF

            @pl.when(cc >= 2)
            def _():
                w_desc(qr).wait()
            if may_issue:
                @pl.when(jnp.logical_and(cc >= 2, cc + 1 < NCHUNK))
                def _():
                    start(cc + 1, qr)
            wait_g(cc, q)
            process(q)
            w_desc(q).start(add=True)

        for q in range(NBUF):
            start(q, q)

        def tri(i3, _):
            c = NBUF * i3
            for q in range(NBUF):
                step(c + q, q)
            return 0
        lax.fori_loop(0, NTRI, tri, 0)

        for q in range(REM):
            cc = NTRI * NBUF + q
            step(cc, q, may_issue=(cc + 1 < NCHUNK))
        for cc in (NCHUNK - 2, NCHUNK - 1):
            w_desc(cc % NBUF).wait()

        plsc.subcore_barrier()
        pltpu.sync_copy(acc_sh.at[pl.ds(sid * STRIPE, STRIPE)],
                        out_hbm.at[cid, pl.ds(sid * STRIPE, STRIPE)])

    return edge


def kernel(x, edge_index, edge_attr, batch, emb_W1, emb_b1, emb_W2, emb_b2,
           rel_w, root_w, conv_b, head_W1, head_b1, head_W2, head_b2):
    N, D = x.shape
    E = edge_index.shape[1]
    R = edge_attr.shape[1]
    H = emb_W1.shape[1]
    OUT = head_W2.shape[1]
    DEPTH = rel_w.shape[0]
    NP = N
    CN = -(-R * N // (NW * 128)) * (NW * 128)

    full = lambda shape: pl.BlockSpec(shape, lambda *_: tuple(0 for _ in shape))
    rowb = pl.BlockSpec((BLK, H), lambda i: (i, 0))
    mblk = pl.BlockSpec((R, BLK, H), lambda i: (0, i, 0))
    pblk0 = pl.BlockSpec((1, BLK, H), lambda i: (0, i, 0))
    pblk1 = pl.BlockSpec((1, BLK, H), lambda i: (1, i, 0))

    src = edge_index[0]
    dst = edge_index[1]
    attr_cm = edge_attr.T.reshape(-1)
    gidx, sidx, cnt_parts = _make_prep(E, R, NP, CN)(src, dst, attr_cm)
    sc_e = _make_scale(E, R * N, CN)(sidx, cnt_parts)

    h, m = pl.pallas_call(
        _emb_m_body,
        grid=(NP // BLK,),
        in_specs=[pl.BlockSpec((BLK, D), lambda i: (i, 0)), full((D, H)),
                  full((1, H)), full((H, H)), full((1, H)), full((R, H, H))],
        out_specs=[rowb, mblk],
        out_shape=[jax.ShapeDtypeStruct((NP, H), jnp.float32),
                   jax.ShapeDtypeStruct((R, NP, H), jnp.float32)],
    )(x, emb_W1, emb_b1.reshape(1, H), emb_W2, emb_b2.reshape(1, H), rel_w[0])

    NA = -(-N // 128) * 128
    edge_call = _make_edge(E, NA, H)

    for l in range(DEPTH):
        parts = edge_call(m.reshape(R * NP, H), gidx, dst, sc_e)

        if l != DEPTH - 1:
            h, m = pl.pallas_call(
                _upd_m_body,
                grid=(NP // BLK,),
                in_specs=[rowb, full((H, H)), full((1, H)), pblk0, pblk1,
                          full((R, H, H))],
                out_specs=[rowb, mblk],
                out_shape=[jax.ShapeDtypeStruct((NP, H), jnp.float32),
                           jax.ShapeDtypeStruct((R, NP, H), jnp.float32)],
            )(h, root_w[l], conv_b[l].reshape(1, H), parts, parts,
              rel_w[l + 1])
        else:
            out = pl.pallas_call(
                _upd_pool_body,
                grid=(NP // BLK,),
                in_specs=[rowb, full((H, H)), full((1, H)), pblk0, pblk1,
                          pl.BlockSpec((1, 1, BLK), lambda i: (i, 0, 0)),
                          full((H, H)), full((1, H)), full((H, OUT)),
                          full((1, OUT))],
                out_specs=full((G, OUT)),
                out_shape=jax.ShapeDtypeStruct((G, OUT), jnp.float32),
                scratch_shapes=[pltpu.VMEM((G, H), jnp.float32)],
            )(h, root_w[l], conv_b[l].reshape(1, H), parts, parts,
              batch.reshape(NP // BLK, 1, BLK),
              head_W1, head_b1.reshape(1, H), head_W2, head_b2.reshape(1, OUT))

    return out


# speedup vs baseline: 1.0937x; 1.0376x over previous
import functools

import jax
import jax.numpy as jnp
from jax import lax
from jax.experimental import pallas as pl
from jax.experimental.pallas import tpu as pltpu
from jax.experimental.pallas import tpu_sc as plsc

G = 64
NC = 2
NS = 16
NW = NC * NS
BLK = 2000
KC = 2000
K = 64


def _mesh():
    return plsc.VectorSubcoreMesh(
        core_axis_name="c", subcore_axis_name="s", num_cores=NC, num_subcores=NS)


def _emb_m_body(x_ref, w1_ref, b1_ref, w2_ref, b2_ref, rw_ref, oh_ref, om_ref):
    t = jnp.dot(x_ref[...], w1_ref[...], preferred_element_type=jnp.float32)
    t = jnp.maximum(t + b1_ref[...], 0.0)
    h = jnp.dot(t, w2_ref[...], preferred_element_type=jnp.float32) + b2_ref[...]
    oh_ref[...] = h
    for r in range(om_ref.shape[0]):
        om_ref[r] = jnp.dot(h, rw_ref[r], preferred_element_type=jnp.float32)


def _upd_m_body(h_ref, w_ref, b_ref, p0_ref, p1_ref, rw_ref, oh_ref, om_ref):
    v = jnp.dot(h_ref[...], w_ref[...], preferred_element_type=jnp.float32)
    v = jnp.maximum(v + b_ref[...] + p0_ref[0] + p1_ref[0], 0.0)
    oh_ref[...] = v
    for r in range(om_ref.shape[0]):
        om_ref[r] = jnp.dot(v, rw_ref[r], preferred_element_type=jnp.float32)


def _upd_pool_body(h_ref, w_ref, b_ref, p0_ref, p1_ref, bt_ref,
                   hw1_ref, hb1_ref, hw2_ref, hb2_ref, o_ref, acc_ref):
    i = pl.program_id(0)

    @pl.when(i == 0)
    def _():
        acc_ref[...] = jnp.zeros_like(acc_ref)

    v = jnp.dot(h_ref[...], w_ref[...], preferred_element_type=jnp.float32)
    v = v + b_ref[...] + p0_ref[0] + p1_ref[0]
    bvec = bt_ref[0]
    oh = (lax.broadcasted_iota(jnp.int32, (G, bvec.shape[1]), 0) == bvec)
    acc_ref[...] += jnp.dot(oh.astype(jnp.float32), v,
                            preferred_element_type=jnp.float32)

    @pl.when(i == pl.num_programs(0) - 1)
    def _():
        p = acc_ref[...]
        t = jnp.maximum(
            jnp.dot(p, hw1_ref[...], preferred_element_type=jnp.float32) + hb1_ref[...], 0.0)
        o_ref[...] = jnp.dot(t, hw2_ref[...], preferred_element_type=jnp.float32) + hb2_ref[...]


def _make_prep(E, R, NP, CN):
    EP = E // NW

    @functools.partial(
        pl.kernel,
        out_type=(jax.ShapeDtypeStruct((E,), jnp.int32),
                  jax.ShapeDtypeStruct((E,), jnp.int32),
                  jax.ShapeDtypeStruct((NW, CN), jnp.float32)),
        mesh=_mesh(),
        compiler_params=pltpu.CompilerParams(needs_layout_passes=False),
        scratch_types=[
            pltpu.VMEM((EP,), jnp.int32),
            pltpu.VMEM((EP,), jnp.int32),
            pltpu.VMEM((EP,), jnp.int32),
            pltpu.VMEM((EP,), jnp.int32),
            pltpu.VMEM((CN,), jnp.float32),
            pltpu.SemaphoreType.DMA,
        ] + [pltpu.VMEM((EP,), jnp.float32) for _ in range(R)],
    )
    def prep(src_hbm, dst_hbm, attr_hbm, gidx_hbm, sidx_hbm, cnt_hbm,
             s_v, d_v, gi_v, si_v, cnt_v, sem, *a_refs):
        cid = lax.axis_index("c")
        sid = lax.axis_index("s")
        w = cid * NS + sid
        ones = jnp.ones((16,), jnp.float32)
        base = w * EP

        descs = ([pltpu.make_async_copy(src_hbm.at[pl.ds(base, EP)], s_v, sem),
                  pltpu.make_async_copy(dst_hbm.at[pl.ds(base, EP)], d_v, sem)]
                 + [pltpu.make_async_copy(
                        attr_hbm.at[pl.ds(r * E + base, EP)], a_refs[r], sem)
                    for r in range(R)])
        for dsc in descs:
            dsc.start()

        def zero(i, _):
            cnt_v[pl.ds(i * 16, 16)] = jnp.zeros((16,), jnp.float32)
            return 0
        lax.fori_loop(0, CN // 16, zero, 0)
        for dsc in descs:
            dsc.wait()

        def grp(j, _):
            off = j * 16
            best = a_refs[0][pl.ds(off, 16)]
            t = jnp.zeros((16,), jnp.int32)
            for r in range(1, R):
                ar = a_refs[r][pl.ds(off, 16)]
                m = ar > best
                t = jnp.where(m, r, t)
                best = jnp.where(m, ar, best)
            sv = s_v[pl.ds(off, 16)]
            dv = d_v[pl.ds(off, 16)]
            gi_v[pl.ds(off, 16)] = t * NP + sv
            si = dv * R + t
            si_v[pl.ds(off, 16)] = si
            plsc.addupdate_scatter(cnt_v, [si], ones)
            return 0
        lax.fori_loop(0, EP // 16, grp, 0)

        pltpu.sync_copy(gi_v, gidx_hbm.at[pl.ds(base, EP)])
        pltpu.sync_copy(si_v, sidx_hbm.at[pl.ds(base, EP)])
        pltpu.sync_copy(cnt_v, cnt_hbm.at[w])

    return prep


def _make_scale(E, NR, CN):
    EP = E // NW
    SW = CN // NW

    @functools.partial(
        pl.kernel,
        out_type=jax.ShapeDtypeStruct((E,), jnp.float32),
        mesh=_mesh(),
        compiler_params=pltpu.CompilerParams(needs_layout_passes=False),
        scratch_types=[
            pltpu.VMEM((KC,), jnp.int32),
            pltpu.VMEM((KC,), jnp.float32),
            pltpu.VMEM((NR,), jnp.float32),
            pltpu.VMEM((NW, SW), jnp.float32),
            pltpu.VMEM((SW,), jnp.float32),
            pltpu.VMEM_SHARED((CN,), jnp.float32),
        ],
    )
    def scale(sidx_hbm, cnt_hbm, sc_hbm, si_v, sc_v, inv_v, parts_v, ist_v, inv_sh):
        cid = lax.axis_index("c")
        sid = lax.axis_index("s")
        w = cid * NS + sid

        for half in range(NW // NS):
            soff = (half * NS + sid) * SW
            pltpu.sync_copy(cnt_hbm.at[pl.ds(0, NW), pl.ds(soff, SW)], parts_v)

            def red(g, _):
                off = g * 16
                s = jnp.zeros((16,), jnp.float32)
                for p in range(NW):
                    s = s + parts_v[p, pl.ds(off, 16)]
                ist_v[pl.ds(off, 16)] = 1.0 / jnp.maximum(s, 1.0)
                return 0
            lax.fori_loop(0, SW // 16, red, 0)
            pltpu.sync_copy(ist_v, inv_sh.at[pl.ds(soff, SW)])
        plsc.subcore_barrier()
        pltpu.sync_copy(inv_sh.at[pl.ds(0, NR)], inv_v)

        def chunk(ci, _):
            base = w * EP + ci * KC
            pltpu.sync_copy(sidx_hbm.at[pl.ds(base, KC)], si_v)

            def grp(j, _):
                off = j * 16
                si = si_v[pl.ds(off, 16)]
                sc_v[pl.ds(off, 16)] = plsc.load_gather(inv_v, [si])
                return 0
            lax.fori_loop(0, KC // 16, grp, 0)

            pltpu.sync_copy(sc_v, sc_hbm.at[pl.ds(base, KC)])
            return 0
        lax.fori_loop(0, EP // KC, chunk, 0)

    return scale


def _make_edge(E, NA, H):
    EP = E // NW
    STRIPE = NA // NS
    NCHUNK = EP // K
    TAIL = EP - NCHUNK * K
    NBUF = 4
    NTRI = NCHUNK // NBUF
    REM = NCHUNK - NTRI * NBUF
    assert TAIL % 16 == 0 and TAIL < K

    @functools.partial(
        pl.kernel,
        out_type=jax.ShapeDtypeStruct((NC, NA, H), jnp.float32),
        mesh=_mesh(),
        compiler_params=pltpu.CompilerParams(needs_layout_passes=False),
        scratch_types=(
            [pltpu.VMEM((K, H), jnp.float32)] * NBUF
            + [pltpu.VMEM((K,), jnp.int32)] * NBUF
            + [pltpu.VMEM((K,), jnp.float32)] * NBUF
            + [pltpu.VMEM((TAIL,), jnp.int32)]
            + [pltpu.VMEM((TAIL,), jnp.float32)]
            + [pltpu.VMEM((EP,), jnp.int32)]
            + [pltpu.VMEM_SHARED((NA, H), jnp.float32)]
            + [pltpu.SemaphoreType.DMA] * (2 * NBUF)
        ),
    )
    def edge(m_hbm, gidx_hbm, dst_hbm, sce_hbm, out_hbm, *scr):
        rows = scr[0:NBUF]
        dbuf = scr[NBUF:2 * NBUF]
        scb = scr[2 * NBUF:3 * NBUF]
        d_t = scr[3 * NBUF]
        sc_t = scr[3 * NBUF + 1]
        gi_all = scr[3 * NBUF + 2]
        acc_sh = scr[3 * NBUF + 3]
        gsem = scr[3 * NBUF + 4:3 * NBUF + 4 + NBUF]
        wsem = scr[3 * NBUF + 4 + NBUF:]
        cid = lax.axis_index("c")
        sid = lax.axis_index("s")
        w = cid * NS + sid
        ebase = w * EP

        def zrow(i, _):
            for c in range(H // 16):
                rows[0][i, pl.ds(c * 16, 16)] = jnp.zeros((16,), jnp.float32)
            return 0
        lax.fori_loop(0, K, zrow, 0)
        for b in range(STRIPE // K):
            pltpu.sync_copy(rows[0], acc_sh.at[pl.ds(sid * STRIPE + b * K, K)])
        rem = STRIPE % K
        if rem:
            pltpu.sync_copy(rows[0].at[pl.ds(0, rem)],
                            acc_sh.at[pl.ds(sid * STRIPE + (STRIPE // K) * K, rem)])
        pltpu.sync_copy(gidx_hbm.at[pl.ds(ebase, EP)], gi_all)
        plsc.subcore_barrier()

        def g_desc(c, p):
            return pltpu.make_async_copy(
                m_hbm.at[gi_all.at[pl.ds(c * K, K)]], rows[p], gsem[p])

        def d_desc(c, p):
            return pltpu.make_async_copy(
                dst_hbm.at[pl.ds(ebase + c * K, K)], dbuf[p], gsem[p])

        def s_desc(c, p):
            return pltpu.make_async_copy(
                sce_hbm.at[pl.ds(ebase + c * K, K)], scb[p], gsem[p])

        def w_desc(p):
            return pltpu.make_async_copy(rows[p], acc_sh.at[dbuf[p]], wsem[p])

        def start(c, p):
            g_desc(c, p).start()
            d_desc(c, p).start()
            s_desc(c, p).start()

        def wait_g(c, p):
            g_desc(c, p).wait()
            d_desc(c, p).wait()
            s_desc(c, p).wait()

        def process(p):
            rb = rows[p]
            sb = scb[p]

            def mj(j2, _):
                off = j2 * 16
                sv = sb[pl.ds(off, 16)]
                for jj in range(16):
                    s = sv[jj]
                    row = off + jj
                    for cc in range(H // 16):
                        rb[row, pl.ds(cc * 16, 16)] = rb[row, pl.ds(cc * 16, 16)] * s
                return 0
            lax.fori_loop(0, K // 16, mj, 0)

        def step(cc, q, may_issue=True):
            qr = (q - 2) % NBUF

            @pl.when(cc >= 2)
            def _():
                w_desc(qr).wait()
            if may_issue:
                @pl.when(jnp.logical_and(cc >= 2, cc + 2 < NCHUNK))
                def _():
                    start(cc + 2, qr)
            wait_g(cc, q)
            process(q)
            w_desc(q).start(add=True)

        for q in range(NBUF):
            start(q, q)

        def tri(i3, _):
            c = NBUF * i3
            for q in range(NBUF):
                step(c + q, q)
            return 0
        lax.fori_loop(0, NTRI, tri, 0)

        for q in range(REM):
            cc = NTRI * NBUF + q
            step(cc, q, may_issue=(cc + 2 < NCHUNK))
        for cc in (NCHUNK - 2, NCHUNK - 1):
            w_desc(cc % NBUF).wait()

        if TAIL:
            tb = NCHUNK * K
            pltpu.make_async_copy(
                dst_hbm.at[pl.ds(ebase + tb, TAIL)], d_t, gsem[0]).start()
            pltpu.make_async_copy(
                sce_hbm.at[pl.ds(ebase + tb, TAIL)], sc_t, gsem[0]).start()
            pltpu.make_async_copy(
                m_hbm.at[gi_all.at[pl.ds(tb, TAIL)]],
                rows[0].at[pl.ds(0, TAIL)], gsem[0]).start()
            pltpu.make_async_copy(
                dst_hbm.at[pl.ds(ebase + tb, TAIL)], d_t, gsem[0]).wait()
            pltpu.make_async_copy(
                sce_hbm.at[pl.ds(ebase + tb, TAIL)], sc_t, gsem[0]).wait()
            pltpu.make_async_copy(
                m_hbm.at[gi_all.at[pl.ds(tb, TAIL)]],
                rows[0].at[pl.ds(0, TAIL)], gsem[0]).wait()
            for j2 in range(TAIL // 16):
                off = j2 * 16
                sv = sc_t[pl.ds(off, 16)]
                for jj in range(16):
                    s = sv[jj]
                    row = off + jj
                    for cc2 in range(H // 16):
                        rows[0][row, pl.ds(cc2 * 16, 16)] = (
                            rows[0][row, pl.ds(cc2 * 16, 16)] * s)
            tw = pltpu.make_async_copy(
                rows[0].at[pl.ds(0, TAIL)], acc_sh.at[d_t], wsem[0])
            tw.start(add=True)
            tw.wait()

        plsc.subcore_barrier()
        pltpu.sync_copy(acc_sh.at[pl.ds(sid * STRIPE, STRIPE)],
                        out_hbm.at[cid, pl.ds(sid * STRIPE, STRIPE)])

    return edge


def kernel(x, edge_index, edge_attr, batch, emb_W1, emb_b1, emb_W2, emb_b2,
           rel_w, root_w, conv_b, head_W1, head_b1, head_W2, head_b2):
    N, D = x.shape
    E = edge_index.shape[1]
    R = edge_attr.shape[1]
    H = emb_W1.shape[1]
    OUT = head_W2.shape[1]
    DEPTH = rel_w.shape[0]
    NP = N
    CN = -(-R * N // (NW * 128)) * (NW * 128)

    full = lambda shape: pl.BlockSpec(shape, lambda *_: tuple(0 for _ in shape))
    rowb = pl.BlockSpec((BLK, H), lambda i: (i, 0))
    mblk = pl.BlockSpec((R, BLK, H), lambda i: (0, i, 0))
    pblk0 = pl.BlockSpec((1, BLK, H), lambda i: (0, i, 0))
    pblk1 = pl.BlockSpec((1, BLK, H), lambda i: (1, i, 0))

    src = edge_index[0]
    dst = edge_index[1]
    attr_cm = edge_attr.T.reshape(-1)
    gidx, sidx, cnt_parts = _make_prep(E, R, NP, CN)(src, dst, attr_cm)
    sc_e = _make_scale(E, R * N, CN)(sidx, cnt_parts)

    h, m = pl.pallas_call(
        _emb_m_body,
        grid=(NP // BLK,),
        in_specs=[pl.BlockSpec((BLK, D), lambda i: (i, 0)), full((D, H)),
                  full((1, H)), full((H, H)), full((1, H)), full((R, H, H))],
        out_specs=[rowb, mblk],
        out_shape=[jax.ShapeDtypeStruct((NP, H), jnp.float32),
                   jax.ShapeDtypeStruct((R, NP, H), jnp.float32)],
    )(x, emb_W1, emb_b1.reshape(1, H), emb_W2, emb_b2.reshape(1, H), rel_w[0])

    NA = -(-N // 128) * 128
    edge_call = _make_edge(E, NA, H)

    for l in range(DEPTH):
        parts = edge_call(m.reshape(R * NP, H), gidx, dst, sc_e)

        if l != DEPTH - 1:
            h, m = pl.pallas_call(
                _upd_m_body,
                grid=(NP // BLK,),
                in_specs=[rowb, full((H, H)), full((1, H)), pblk0, pblk1,
                          full((R, H, H))],
                out_specs=[rowb, mblk],
                out_shape=[jax.ShapeDtypeStruct((NP, H), jnp.float32),
                           jax.ShapeDtypeStruct((R, NP, H), jnp.float32)],
            )(h, root_w[l], conv_b[l].reshape(1, H), parts, parts,
              rel_w[l + 1])
        else:
            out = pl.pallas_call(
                _upd_pool_body,
                grid=(NP // BLK,),
                in_specs=[rowb, full((H, H)), full((1, H)), pblk0, pblk1,
                          pl.BlockSpec((1, 1, BLK), lambda i: (i, 0, 0)),
                          full((H, H)), full((1, H)), full((H, OUT)),
                          full((1, OUT))],
                out_specs=full((G, OUT)),
                out_shape=jax.ShapeDtypeStruct((G, OUT), jnp.float32),
                scratch_shapes=[pltpu.VMEM((G, H), jnp.float32)],
            )(h, root_w[l], conv_b[l].reshape(1, H), parts, parts,
              batch.reshape(NP // BLK, 1, BLK),
              head_W1, head_b1.reshape(1, H), head_W2, head_b2.reshape(1, OUT))

    return out
